# Initial kernel scaffold; baseline (speedup 1.0000x reference)
import jax
import jax.numpy as jnp
from jax.experimental import pallas as pl


def kernel(x, edge_index, edge_weight, W1, b1, g1, be1, W2, b2, g2, be2, W3, b3):
    raise NotImplementedError("write your pallas kernel here")


# speedup vs baseline: 9.8181x; 9.8181x over previous
import functools

import jax
import jax.numpy as jnp
from jax import lax
from jax.experimental import pallas as pl
from jax.experimental.pallas import tpu as pltpu
from jax.experimental.pallas import tpu_sc as plsc

N = 10000
E = 320000
D = 128
NC = 2
NS = 16
L = 16
K = 125
CH = E // (NC * NS * K)
NP = 10240
RPT = NP // NS

_mesh = plsc.VectorSubcoreMesh(
    core_axis_name="c", subcore_axis_name="s", num_cores=NC, num_subcores=NS)


@functools.partial(
    pl.kernel,
    out_type=jax.ShapeDtypeStruct((NC, NP, L), jnp.float32),
    mesh=_mesh,
    compiler_params=pltpu.CompilerParams(use_tc_tiling_on_sc=False),
    scratch_types=[
        pltpu.VMEM_SHARED((NP, L), jnp.float32),
        pltpu.VMEM((CH, K), jnp.int32),
        pltpu.VMEM((K, L), jnp.float32),
    ],
)
def _sc_deg(wrep, zer16, dstr, out, acc, dst_v, wrep_v):
    c = lax.axis_index("c")
    s = lax.axis_index("s")
    pltpu.sync_copy(zer16.at[pl.ds(s * RPT, RPT)], acc.at[pl.ds(s * RPT, RPT)])
    pltpu.sync_copy(dstr.at[c, s], dst_v)
    plsc.subcore_barrier()

    def chunk(j, carry):
        pltpu.sync_copy(wrep.at[c, s, j], wrep_v)
        pltpu.sync_copy(wrep_v, acc.at[dst_v.at[j]], add=True)
        return carry

    lax.fori_loop(0, CH, chunk, 0)
    plsc.subcore_barrier()
    pltpu.sync_copy(acc.at[pl.ds(s * RPT, RPT)], out.at[c, pl.ds(s * RPT, RPT)])


@functools.partial(
    pl.kernel,
    out_type=jax.ShapeDtypeStruct((NC, NP, D), jnp.float32),
    mesh=_mesh,
    compiler_params=pltpu.CompilerParams(use_tc_tiling_on_sc=False),
    scratch_types=[
        pltpu.VMEM_SHARED((NP, D), jnp.float32),
        pltpu.VMEM((CH, K), jnp.int32),
        pltpu.VMEM((CH, K), jnp.int32),
        pltpu.VMEM((K, L), jnp.float32),
        pltpu.VMEM((K, D), jnp.float32),
        pltpu.SemaphoreType.DMA,
    ],
)
def _sc_agg(hs, zer, srcr, dstr, wrep, out, acc, src_v, dst_v, wrep_v, rows_v,
            sem):
    c = lax.axis_index("c")
    s = lax.axis_index("s")
    @pl.when(c == 0)
    def _():
        pltpu.sync_copy(hs.at[pl.ds(s * RPT, RPT)], acc.at[pl.ds(s * RPT, RPT)])

    @pl.when(c != 0)
    def _():
        pltpu.sync_copy(zer.at[pl.ds(s * RPT, RPT)], acc.at[pl.ds(s * RPT, RPT)])

    pltpu.sync_copy(srcr.at[c, s], src_v)
    pltpu.sync_copy(dstr.at[c, s], dst_v)
    plsc.subcore_barrier()

    def chunk(j, carry):
        pltpu.sync_copy(wrep.at[c, s, j], wrep_v)
        pltpu.async_copy(hs.at[src_v.at[j]], rows_v, sem).wait()

        def edge(e, c2):
            sp = wrep_v[e, :]
            for k in range(D // L):
                rows_v[e, pl.ds(k * L, L)] = rows_v[e, pl.ds(k * L, L)] * sp
            return c2

        lax.fori_loop(0, K, edge, 0)
        pltpu.sync_copy(rows_v, acc.at[dst_v.at[j]], add=True)
        return carry

    lax.fori_loop(0, CH, chunk, 0)
    plsc.subcore_barrier()
    pltpu.sync_copy(acc.at[pl.ds(s * RPT, RPT)], out.at[c, pl.ds(s * RPT, RPT)])


def _tc1_body(degp, x, w1, dinv, hs):
    deg = degp[0, 0:N, 0:1] + degp[1, 0:N, 0:1] + 1.0
    dv = lax.rsqrt(deg)
    dinv[...] = dv
    hs[0:N, :] = dv * jnp.dot(x[...], w1[...], preferred_element_type=jnp.float32)
    hs[N:NP, :] = jnp.zeros((NP - N, D), jnp.float32)


def _tc_mid_body(p, dinv, b, g, be, w, hs):
    dv = dinv[...]
    t = dv * (p[0, 0:N, :] + p[1, 0:N, :]) + b[...][None, :]
    r = jnp.maximum(t, 0.0)
    mu = jnp.mean(r, axis=0, keepdims=True)
    var = jnp.mean(jnp.square(r - mu), axis=0, keepdims=True)
    bn = (r - mu) * lax.rsqrt(var + 1e-5) * g[...][None, :] + be[...][None, :]
    hs[0:N, :] = dv * jnp.dot(bn, w[...], preferred_element_type=jnp.float32)
    hs[N:NP, :] = jnp.zeros((NP - N, D), jnp.float32)


def _tc_final_body(p, dinv, b, out):
    out[...] = dinv[...] * (p[0, 0:N, :] + p[1, 0:N, :]) + b[...][None, :]


_f32 = jnp.float32
_tc1 = pl.pallas_call(
    _tc1_body,
    out_shape=(jax.ShapeDtypeStruct((N, 1), _f32),
               jax.ShapeDtypeStruct((NP, D), _f32)))
_tc_mid = pl.pallas_call(
    _tc_mid_body, out_shape=jax.ShapeDtypeStruct((NP, D), _f32))
_tc_final = pl.pallas_call(
    _tc_final_body, out_shape=jax.ShapeDtypeStruct((N, D), _f32))


def kernel(x, edge_index, edge_weight, W1, b1, g1, be1, W2, b2, g2, be2, W3,
           b3):
    src = edge_index[0].reshape(NC, NS, CH, K)
    dst = edge_index[1].reshape(NC, NS, CH, K)
    wrep = jnp.broadcast_to(edge_weight[:, None], (E, L)).reshape(
        NC, NS, CH, K, L)
    zeros = jnp.zeros((NP, D), _f32)
    zeros16 = jnp.zeros((NP, L), _f32)

    degp = _sc_deg(wrep, zeros16, dst)
    dinv, hs = _tc1(degp, x, W1)
    p = _sc_agg(hs, zeros, src, dst, wrep)
    hs = _tc_mid(p, dinv, b1, g1, be1, W2)
    p = _sc_agg(hs, zeros, src, dst, wrep)
    hs = _tc_mid(p, dinv, b2, g2, be2, W3)
    p = _sc_agg(hs, zeros, src, dst, wrep)
    return _tc_final(p, dinv, b3)
